# baseline (speedup 1.0000x reference)
import functools

import jax
import jax.numpy as jnp
from jax import lax
from jax.experimental import pallas as pl
from jax.experimental.pallas import tpu as pltpu
from jax.experimental.pallas import tpu_sc as plsc

B, C, M, W = 16, 4, 32, 16
D_MODEL = 256
D_K = 64
N = B * C * M
TOK = N * W

NC, NS = 2, 16
NW = NC * NS
CHUNK = 128
NBUF = 3


def _sc_gather(table, ids, num_cores=NC):
  ntok = ids.shape[0]
  nw = num_cores * NS
  tpw = ntok // nw
  cpw = tpw // CHUNK
  mesh = plsc.VectorSubcoreMesh(core_axis_name="c", subcore_axis_name="s",
                                num_cores=num_cores)

  @functools.partial(
      pl.kernel,
      mesh=mesh,
      out_type=jax.ShapeDtypeStruct((ntok, D_MODEL), jnp.float32),
      scratch_types=[
          pltpu.VMEM((tpw,), jnp.int32),
          pltpu.VMEM((NBUF, CHUNK, D_MODEL), jnp.float32),
          pltpu.SemaphoreType.DMA,
          pltpu.SemaphoreType.DMA,
          pltpu.SemaphoreType.DMA,
          pltpu.SemaphoreType.DMA,
          pltpu.SemaphoreType.DMA,
          pltpu.SemaphoreType.DMA,
      ],
  )
  def gather_kernel(table_hbm, ids_hbm, out_hbm, idx_v, rows_v,
                    g0, g1, g2, w0, w1, w2):
    CPW = cpw
    wid = lax.axis_index("s") * num_cores + lax.axis_index("c")
    base = wid * tpw
    pltpu.sync_copy(ids_hbm.at[pl.ds(base, tpw)], idx_v)
    gsems = [g0, g1, g2]
    wsems = [w0, w1, w2]

    def gstart(g):
      b = g % NBUF
      return pltpu.async_copy(
          table_hbm.at[idx_v.at[pl.ds(g * CHUNK, CHUNK)]],
          rows_v.at[b], gsems[b])

    gh = [None] * CPW
    wh = [None] * CPW
    waited = [False] * CPW
    for g in range(min(NBUF - 1, CPW)):
      gh[g] = gstart(g)
    for g in range(CPW):
      b = g % NBUF
      gh[g].wait()
      wh[g] = pltpu.async_copy(
          rows_v.at[b], out_hbm.at[pl.ds(base + g * CHUNK, CHUNK)], wsems[b])
      n = g + NBUF - 1
      if n < CPW:
        prev = n - NBUF
        if prev >= 0:
          wh[prev].wait()
          waited[prev] = True
        gh[n] = gstart(n)
    for g in range(CPW):
      if not waited[g]:
        wh[g].wait()

  return gather_kernel(table, ids)


NPB = N // B


def _tc_body(bags_ref, h_ref, wp_ref, bp_ref, wq_ref, bq_ref, v_ref,
             lens_ref, szs_ref, out_ref):
  flat = bags_ref[...]
  pre = jnp.dot(flat.astype(jnp.bfloat16), wp_ref[...].astype(jnp.bfloat16),
                preferred_element_type=jnp.float32) + bp_ref[...]
  q = jnp.dot(h_ref[0], wq_ref[...],
              preferred_element_type=jnp.float32) + bq_ref[...]
  e = jnp.dot(jnp.tanh(pre + q), v_ref[...],
              preferred_element_type=jnp.float32)
  p = jnp.exp(e)
  lens = lens_ref[...]
  ps = []
  denom = None
  for w in range(W):
    pw = p[w * NPB:(w + 1) * NPB] * (lens > float(w)).astype(jnp.float32)
    ps.append(pw)
    denom = pw if w == 0 else denom + pw
  mpos = lax.broadcasted_iota(jnp.int32, (NPB, 1), 0) % M
  nmask = (mpos.astype(jnp.float32) < szs_ref[...]).astype(jnp.float32)
  scale = nmask / denom
  ctx = ps[0] * scale * flat[0:NPB]
  for w in range(1, W):
    ctx = ctx + (ps[w] * scale) * flat[w * NPB:(w + 1) * NPB]
  out_ref[...] = ctx


def _tc_attention(bags_t, hidden, W_pre, b_pre, W_q, b_q, v_att,
                  lens_node, sizes_node):
  nb = hidden.shape[0]
  return pl.pallas_call(
      _tc_body,
      grid=(nb,),
      in_specs=[
          pl.BlockSpec((W * NPB, D_MODEL), lambda i: (i, 0)),
          pl.BlockSpec((1, 1, D_MODEL), lambda i: (i, 0, 0)),
          pl.BlockSpec((D_MODEL, D_K), lambda i: (0, 0)),
          pl.BlockSpec((1, D_K), lambda i: (0, 0)),
          pl.BlockSpec((D_MODEL, D_K), lambda i: (0, 0)),
          pl.BlockSpec((1, D_K), lambda i: (0, 0)),
          pl.BlockSpec((D_K, 1), lambda i: (0, 0)),
          pl.BlockSpec((NPB, 1), lambda i: (i, 0)),
          pl.BlockSpec((NPB, 1), lambda i: (i, 0)),
      ],
      out_specs=pl.BlockSpec((NPB, D_MODEL), lambda i: (i, 0)),
      out_shape=jax.ShapeDtypeStruct((nb * NPB, D_MODEL), jnp.float32),
  )(bags_t, hidden, W_pre, b_pre, W_q, b_q, v_att, lens_node, sizes_node)


NSPLIT = 2


def kernel(con_hidden, emb_table, W_pre, b_pre, W_q, b_q, v_att,
           token_ids, node_lengths, node_sizes):
  hidden = jnp.concatenate([con_hidden[0], con_hidden[1]], axis=1)
  ids3 = token_ids.reshape(B, NPB, W).swapaxes(1, 2)
  lens_node = node_lengths.astype(jnp.float32).reshape(N, 1)
  sizes_node = jnp.repeat(node_sizes, M).astype(jnp.float32).reshape(N, 1)
  h3 = hidden.reshape(B, 1, D_MODEL)
  bp2, bq2, v2 = b_pre.reshape(1, D_K), b_q.reshape(1, D_K), v_att.reshape(D_K, 1)
  nh = N // NSPLIT
  bh = B // NSPLIT
  bags = [
      _sc_gather(emb_table, ids3[k * bh:(k + 1) * bh].reshape(bh * W * NPB))
      for k in range(NSPLIT)
  ]
  ctxs = [
      _tc_attention(bags[k], h3[k * bh:(k + 1) * bh], W_pre, bp2, W_q, bq2, v2,
                    lens_node[k * nh:(k + 1) * nh],
                    sizes_node[k * nh:(k + 1) * nh])
      for k in range(NSPLIT)
  ]
  ctx = jnp.concatenate(ctxs, axis=0)
  return (ctx.reshape(B, C, M, D_MODEL), hidden)


# speedup vs baseline: 1.0550x; 1.0550x over previous
import functools

import jax
import jax.numpy as jnp
from jax import lax
from jax.experimental import pallas as pl
from jax.experimental.pallas import tpu as pltpu
from jax.experimental.pallas import tpu_sc as plsc

B, C, M, W = 16, 4, 32, 16
D_MODEL = 256
D_K = 64
N = B * C * M
TOK = N * W

NC, NS = 2, 16
NW = NC * NS
CHUNK = 128
NBUF = 3


def _sc_gather(table, ids, num_cores=NC):
  ntok = ids.shape[0]
  nw = num_cores * NS
  tpw = ntok // nw
  cpw = tpw // CHUNK
  mesh = plsc.VectorSubcoreMesh(core_axis_name="c", subcore_axis_name="s",
                                num_cores=num_cores)

  @functools.partial(
      pl.kernel,
      mesh=mesh,
      out_type=jax.ShapeDtypeStruct((ntok, D_MODEL), jnp.float32),
      scratch_types=[
          pltpu.VMEM((tpw,), jnp.int32),
          pltpu.VMEM((NBUF, CHUNK, D_MODEL), jnp.float32),
          pltpu.SemaphoreType.DMA,
          pltpu.SemaphoreType.DMA,
          pltpu.SemaphoreType.DMA,
          pltpu.SemaphoreType.DMA,
          pltpu.SemaphoreType.DMA,
          pltpu.SemaphoreType.DMA,
      ],
  )
  def gather_kernel(table_hbm, ids_hbm, out_hbm, idx_v, rows_v,
                    g0, g1, g2, w0, w1, w2):
    CPW = cpw
    wid = lax.axis_index("s") * num_cores + lax.axis_index("c")
    base = wid * tpw
    pltpu.sync_copy(ids_hbm.at[pl.ds(base, tpw)], idx_v)
    gsems = [g0, g1, g2]
    wsems = [w0, w1, w2]

    def gstart(g):
      b = g % NBUF
      return pltpu.async_copy(
          table_hbm.at[idx_v.at[pl.ds(g * CHUNK, CHUNK)]],
          rows_v.at[b], gsems[b])

    gh = [None] * CPW
    wh = [None] * CPW
    for g in range(min(NBUF, CPW)):
      gh[g] = gstart(g)
    for g in range(CPW):
      b = g % NBUF
      gh[g].wait()
      wh[g] = pltpu.async_copy(
          rows_v.at[b], out_hbm.at[pl.ds(base + g * CHUNK, CHUNK)], wsems[b])
      if g + NBUF < CPW:
        wh[g].wait()
        gh[g + NBUF] = gstart(g + NBUF)
    for g in range(max(0, CPW - NBUF), CPW):
      wh[g].wait()

  return gather_kernel(table, ids)


NPB = N // B


BPG = 2


def _tc_body(bags_ref, h_ref, wp_ref, bp_ref, wq_ref, bq_ref, v_ref,
             lens_ref, szs_ref, out_ref):
  flat = bags_ref[...]
  pre = jnp.dot(flat.astype(jnp.bfloat16), wp_ref[...].astype(jnp.bfloat16),
                preferred_element_type=jnp.float32) + bp_ref[...]
  tpb = W * NPB
  for bb in range(BPG):
    q = jnp.dot(h_ref[bb], wq_ref[...],
                preferred_element_type=jnp.float32) + bq_ref[...]
    e = jnp.dot(jnp.tanh(pre[bb * tpb:(bb + 1) * tpb] + q), v_ref[...],
                preferred_element_type=jnp.float32)
    p = jnp.exp(e)
    lens = lens_ref[bb * NPB:(bb + 1) * NPB]
    ps = []
    denom = None
    for w in range(W):
      pw = p[w * NPB:(w + 1) * NPB] * (lens > float(w)).astype(jnp.float32)
      ps.append(pw)
      denom = pw if w == 0 else denom + pw
    mpos = lax.broadcasted_iota(jnp.int32, (NPB, 1), 0) % M
    nmask = (mpos.astype(jnp.float32) < szs_ref[bb * NPB:(bb + 1) * NPB]
             ).astype(jnp.float32)
    scale = nmask / denom
    base = bb * tpb
    ctx = ps[0] * scale * flat[base:base + NPB]
    for w in range(1, W):
      ctx = ctx + (ps[w] * scale) * flat[base + w * NPB:base + (w + 1) * NPB]
    out_ref[bb * NPB:(bb + 1) * NPB, :] = ctx


def _tc_attention(bags_t, hidden, W_pre, b_pre, W_q, b_q, v_att,
                  lens_node, sizes_node):
  nb = hidden.shape[0]
  return pl.pallas_call(
      _tc_body,
      grid=(nb // BPG,),
      in_specs=[
          pl.BlockSpec((BPG * W * NPB, D_MODEL), lambda i: (i, 0)),
          pl.BlockSpec((BPG, 1, D_MODEL), lambda i: (i, 0, 0)),
          pl.BlockSpec((D_MODEL, D_K), lambda i: (0, 0)),
          pl.BlockSpec((1, D_K), lambda i: (0, 0)),
          pl.BlockSpec((D_MODEL, D_K), lambda i: (0, 0)),
          pl.BlockSpec((1, D_K), lambda i: (0, 0)),
          pl.BlockSpec((D_K, 1), lambda i: (0, 0)),
          pl.BlockSpec((BPG * NPB, 1), lambda i: (i, 0)),
          pl.BlockSpec((BPG * NPB, 1), lambda i: (i, 0)),
      ],
      out_specs=pl.BlockSpec((BPG * NPB, D_MODEL), lambda i: (i, 0)),
      out_shape=jax.ShapeDtypeStruct((nb * NPB, D_MODEL), jnp.float32),
  )(bags_t, hidden, W_pre, b_pre, W_q, b_q, v_att, lens_node, sizes_node)


NSPLIT = 2


def kernel(con_hidden, emb_table, W_pre, b_pre, W_q, b_q, v_att,
           token_ids, node_lengths, node_sizes):
  hidden = jnp.concatenate([con_hidden[0], con_hidden[1]], axis=1)
  ids3 = token_ids.reshape(B, NPB, W).swapaxes(1, 2)
  lens_node = node_lengths.astype(jnp.float32).reshape(N, 1)
  sizes_node = jnp.repeat(node_sizes, M).astype(jnp.float32).reshape(N, 1)
  h3 = hidden.reshape(B, 1, D_MODEL)
  bp2, bq2, v2 = b_pre.reshape(1, D_K), b_q.reshape(1, D_K), v_att.reshape(D_K, 1)
  nh = N // NSPLIT
  bh = B // NSPLIT
  bags = [
      _sc_gather(emb_table, ids3[k * bh:(k + 1) * bh].reshape(bh * W * NPB))
      for k in range(NSPLIT)
  ]
  ctxs = [
      _tc_attention(bags[k], h3[k * bh:(k + 1) * bh], W_pre, bp2, W_q, bq2, v2,
                    lens_node[k * nh:(k + 1) * nh],
                    sizes_node[k * nh:(k + 1) * nh])
      for k in range(NSPLIT)
  ]
  ctx = jnp.concatenate(ctxs, axis=0)
  return (ctx.reshape(B, C, M, D_MODEL), hidden)


# speedup vs baseline: 1.0638x; 1.0083x over previous
import functools

import jax
import jax.numpy as jnp
from jax import lax
from jax.experimental import pallas as pl
from jax.experimental.pallas import tpu as pltpu
from jax.experimental.pallas import tpu_sc as plsc

B, C, M, W = 16, 4, 32, 16
D_MODEL = 256
D_K = 64
N = B * C * M
TOK = N * W

NC, NS = 2, 16
NW = NC * NS
CHUNK = 128
NBUF = 3


def _sc_gather(table, ids, num_cores=NC):
  ntok = ids.shape[0]
  nw = num_cores * NS
  tpw = ntok // nw
  cpw = tpw // CHUNK
  mesh = plsc.VectorSubcoreMesh(core_axis_name="c", subcore_axis_name="s",
                                num_cores=num_cores)

  @functools.partial(
      pl.kernel,
      mesh=mesh,
      out_type=jax.ShapeDtypeStruct((ntok, D_MODEL), jnp.float32),
      scratch_types=[
          pltpu.VMEM((tpw,), jnp.int32),
          pltpu.VMEM((NBUF, CHUNK, D_MODEL), jnp.float32),
          pltpu.SemaphoreType.DMA,
          pltpu.SemaphoreType.DMA,
          pltpu.SemaphoreType.DMA,
          pltpu.SemaphoreType.DMA,
          pltpu.SemaphoreType.DMA,
          pltpu.SemaphoreType.DMA,
      ],
  )
  def gather_kernel(table_hbm, ids_hbm, out_hbm, idx_v, rows_v,
                    g0, g1, g2, w0, w1, w2):
    CPW = cpw
    wid = lax.axis_index("s") * num_cores + lax.axis_index("c")
    base = wid * tpw
    pltpu.sync_copy(ids_hbm.at[pl.ds(base, tpw)], idx_v)
    gsems = [g0, g1, g2]
    wsems = [w0, w1, w2]

    def gstart(g):
      b = g % NBUF
      return pltpu.async_copy(
          table_hbm.at[idx_v.at[pl.ds(g * CHUNK, CHUNK)]],
          rows_v.at[b], gsems[b])

    gh = [None] * CPW
    wh = [None] * CPW
    for g in range(min(NBUF, CPW)):
      gh[g] = gstart(g)
    for g in range(CPW):
      b = g % NBUF
      gh[g].wait()
      wh[g] = pltpu.async_copy(
          rows_v.at[b], out_hbm.at[pl.ds(base + g * CHUNK, CHUNK)], wsems[b])
      if g + NBUF < CPW:
        wh[g].wait()
        gh[g + NBUF] = gstart(g + NBUF)
    for g in range(max(0, CPW - NBUF), CPW):
      wh[g].wait()

  return gather_kernel(table, ids)


NPB = N // B


BPG = 4


def _tc_body(bags_ref, h_ref, wp_ref, bp_ref, wq_ref, bq_ref, v_ref,
             lens_ref, szs_ref, out_ref):
  flat = bags_ref[...]
  pre = jnp.dot(flat.astype(jnp.bfloat16), wp_ref[...].astype(jnp.bfloat16),
                preferred_element_type=jnp.float32) + bp_ref[...]
  tpb = W * NPB
  for bb in range(BPG):
    q = jnp.dot(h_ref[bb], wq_ref[...],
                preferred_element_type=jnp.float32) + bq_ref[...]
    e = jnp.dot(jnp.tanh(pre[bb * tpb:(bb + 1) * tpb] + q), v_ref[...],
                preferred_element_type=jnp.float32)
    p = jnp.exp(e)
    lens = lens_ref[bb * NPB:(bb + 1) * NPB]
    ps = []
    denom = None
    for w in range(W):
      pw = p[w * NPB:(w + 1) * NPB] * (lens > float(w)).astype(jnp.float32)
      ps.append(pw)
      denom = pw if w == 0 else denom + pw
    mpos = lax.broadcasted_iota(jnp.int32, (NPB, 1), 0) % M
    nmask = (mpos.astype(jnp.float32) < szs_ref[bb * NPB:(bb + 1) * NPB]
             ).astype(jnp.float32)
    scale = nmask / denom
    base = bb * tpb
    ctx = ps[0] * scale * flat[base:base + NPB]
    for w in range(1, W):
      ctx = ctx + (ps[w] * scale) * flat[base + w * NPB:base + (w + 1) * NPB]
    out_ref[bb * NPB:(bb + 1) * NPB, :] = ctx


def _tc_attention(bags_t, hidden, W_pre, b_pre, W_q, b_q, v_att,
                  lens_node, sizes_node):
  nb = hidden.shape[0]
  return pl.pallas_call(
      _tc_body,
      grid=(nb // BPG,),
      in_specs=[
          pl.BlockSpec((BPG * W * NPB, D_MODEL), lambda i: (i, 0)),
          pl.BlockSpec((BPG, 1, D_MODEL), lambda i: (i, 0, 0)),
          pl.BlockSpec((D_MODEL, D_K), lambda i: (0, 0)),
          pl.BlockSpec((1, D_K), lambda i: (0, 0)),
          pl.BlockSpec((D_MODEL, D_K), lambda i: (0, 0)),
          pl.BlockSpec((1, D_K), lambda i: (0, 0)),
          pl.BlockSpec((D_K, 1), lambda i: (0, 0)),
          pl.BlockSpec((BPG * NPB, 1), lambda i: (i, 0)),
          pl.BlockSpec((BPG * NPB, 1), lambda i: (i, 0)),
      ],
      out_specs=pl.BlockSpec((BPG * NPB, D_MODEL), lambda i: (i, 0)),
      out_shape=jax.ShapeDtypeStruct((nb * NPB, D_MODEL), jnp.float32),
  )(bags_t, hidden, W_pre, b_pre, W_q, b_q, v_att, lens_node, sizes_node)


NSPLIT = 2


def kernel(con_hidden, emb_table, W_pre, b_pre, W_q, b_q, v_att,
           token_ids, node_lengths, node_sizes):
  hidden = jnp.concatenate([con_hidden[0], con_hidden[1]], axis=1)
  ids3 = token_ids.reshape(B, NPB, W).swapaxes(1, 2)
  lens_node = node_lengths.astype(jnp.float32).reshape(N, 1)
  sizes_node = jnp.repeat(node_sizes, M).astype(jnp.float32).reshape(N, 1)
  h3 = hidden.reshape(B, 1, D_MODEL)
  bp2, bq2, v2 = b_pre.reshape(1, D_K), b_q.reshape(1, D_K), v_att.reshape(D_K, 1)
  nh = N // NSPLIT
  bh = B // NSPLIT
  bags = [
      _sc_gather(emb_table, ids3[k * bh:(k + 1) * bh].reshape(bh * W * NPB))
      for k in range(NSPLIT)
  ]
  ctxs = [
      _tc_attention(bags[k], h3[k * bh:(k + 1) * bh], W_pre, bp2, W_q, bq2, v2,
                    lens_node[k * nh:(k + 1) * nh],
                    sizes_node[k * nh:(k + 1) * nh])
      for k in range(NSPLIT)
  ]
  ctx = jnp.concatenate(ctxs, axis=0)
  return (ctx.reshape(B, C, M, D_MODEL), hidden)


# speedup vs baseline: 1.1170x; 1.0500x over previous
import functools

import jax
import jax.numpy as jnp
from jax import lax
from jax.experimental import pallas as pl
from jax.experimental.pallas import tpu as pltpu
from jax.experimental.pallas import tpu_sc as plsc

B, C, M, W = 16, 4, 32, 16
D_MODEL = 256
D_K = 64
N = B * C * M
TOK = N * W

NC, NS = 2, 16
NW = NC * NS
CHUNK = 128
NBUF = 3


def _sc_gather(table, ids, num_cores=NC):
  ntok = ids.shape[0]
  nw = num_cores * NS
  tpw = ntok // nw
  cpw = tpw // CHUNK
  mesh = plsc.VectorSubcoreMesh(core_axis_name="c", subcore_axis_name="s",
                                num_cores=num_cores)

  @functools.partial(
      pl.kernel,
      mesh=mesh,
      out_type=jax.ShapeDtypeStruct((ntok, D_MODEL), jnp.float32),
      scratch_types=[
          pltpu.VMEM((tpw,), jnp.int32),
          pltpu.VMEM((NBUF, CHUNK, D_MODEL), jnp.float32),
          pltpu.SemaphoreType.DMA,
          pltpu.SemaphoreType.DMA,
          pltpu.SemaphoreType.DMA,
          pltpu.SemaphoreType.DMA,
          pltpu.SemaphoreType.DMA,
          pltpu.SemaphoreType.DMA,
      ],
  )
  def gather_kernel(table_hbm, ids_hbm, out_hbm, idx_v, rows_v,
                    g0, g1, g2, w0, w1, w2):
    CPW = cpw
    wid = lax.axis_index("s") * num_cores + lax.axis_index("c")
    base = wid * tpw
    pltpu.sync_copy(ids_hbm.at[pl.ds(base, tpw)], idx_v)
    gsems = [g0, g1, g2]
    wsems = [w0, w1, w2]

    def gstart(g):
      b = g % NBUF
      return pltpu.async_copy(
          table_hbm.at[idx_v.at[pl.ds(g * CHUNK, CHUNK)]],
          rows_v.at[b], gsems[b])

    gh = [None] * CPW
    wh = [None] * CPW
    for g in range(min(NBUF, CPW)):
      gh[g] = gstart(g)
    for g in range(CPW):
      b = g % NBUF
      gh[g].wait()
      wh[g] = pltpu.async_copy(
          rows_v.at[b], out_hbm.at[pl.ds(base + g * CHUNK, CHUNK)], wsems[b])
      if g + NBUF < CPW:
        wh[g].wait()
        gh[g + NBUF] = gstart(g + NBUF)
    for g in range(max(0, CPW - NBUF), CPW):
      wh[g].wait()

  return gather_kernel(table, ids)


NPB = N // B


BPG = 4


def _tc_body(bags_ref, h_ref, wp_ref, bp_ref, wq_ref, bq_ref, v_ref,
             lens_ref, szs_ref, out_ref):
  flat = bags_ref[...]
  pre = jnp.dot(flat.astype(jnp.bfloat16), wp_ref[...].astype(jnp.bfloat16),
                preferred_element_type=jnp.float32) + bp_ref[...]
  tpb = W * NPB
  for bb in range(BPG):
    q = jnp.dot(h_ref[bb], wq_ref[...],
                preferred_element_type=jnp.float32) + bq_ref[...]
    e = jnp.dot(jnp.tanh(pre[bb * tpb:(bb + 1) * tpb] + q), v_ref[...],
                preferred_element_type=jnp.float32)
    p = jnp.exp(e)
    lens = lens_ref[bb * NPB:(bb + 1) * NPB]
    ps = []
    denom = None
    for w in range(W):
      pw = p[w * NPB:(w + 1) * NPB] * (lens > float(w)).astype(jnp.float32)
      ps.append(pw)
      denom = pw if w == 0 else denom + pw
    mpos = lax.broadcasted_iota(jnp.int32, (NPB, 1), 0) % M
    nmask = (mpos.astype(jnp.float32) < szs_ref[bb * NPB:(bb + 1) * NPB]
             ).astype(jnp.float32)
    scale = nmask / denom
    base = bb * tpb
    ctx = ps[0] * scale * flat[base:base + NPB]
    for w in range(1, W):
      ctx = ctx + (ps[w] * scale) * flat[base + w * NPB:base + (w + 1) * NPB]
    out_ref[bb * NPB:(bb + 1) * NPB, :] = ctx


def _tc_attention(bags_t, hidden, W_pre, b_pre, W_q, b_q, v_att,
                  lens_node, sizes_node, off_b, ctx_init=None):
  nb = bags_t.shape[0] // (W * NPB)
  in_specs = [
      pl.BlockSpec((BPG * W * NPB, D_MODEL), lambda i: (i, 0)),
      pl.BlockSpec((BPG, 1, D_MODEL), lambda i: (i + off_b, 0, 0)),
      pl.BlockSpec((D_MODEL, D_K), lambda i: (0, 0)),
      pl.BlockSpec((1, D_K), lambda i: (0, 0)),
      pl.BlockSpec((D_MODEL, D_K), lambda i: (0, 0)),
      pl.BlockSpec((1, D_K), lambda i: (0, 0)),
      pl.BlockSpec((D_K, 1), lambda i: (0, 0)),
      pl.BlockSpec((BPG * NPB, 1), lambda i: (i + off_b, 0)),
      pl.BlockSpec((BPG * NPB, 1), lambda i: (i + off_b, 0)),
  ]
  args = [bags_t, hidden, W_pre, b_pre, W_q, b_q, v_att,
          lens_node, sizes_node]
  body = _tc_body
  aliases = {}
  if ctx_init is not None:
    in_specs.append(pl.BlockSpec(memory_space=pltpu.MemorySpace.HBM))
    args.append(ctx_init)
    aliases = {9: 0}
    body = lambda *refs: _tc_body(*refs[:9], refs[10])
  return pl.pallas_call(
      body,
      grid=(nb // BPG,),
      in_specs=in_specs,
      out_specs=pl.BlockSpec((BPG * NPB, D_MODEL), lambda i: (i + off_b, 0)),
      out_shape=jax.ShapeDtypeStruct((N, D_MODEL), jnp.float32),
      input_output_aliases=aliases,
  )(*args)


NSPLIT = 2


def kernel(con_hidden, emb_table, W_pre, b_pre, W_q, b_q, v_att,
           token_ids, node_lengths, node_sizes):
  hidden = jnp.concatenate([con_hidden[0], con_hidden[1]], axis=1)
  ids3 = token_ids.reshape(B, NPB, W).swapaxes(1, 2)
  lens_node = node_lengths.astype(jnp.float32).reshape(N, 1)
  sizes_node = jnp.repeat(node_sizes, M).astype(jnp.float32).reshape(N, 1)
  h3 = hidden.reshape(B, 1, D_MODEL)
  bp2, bq2, v2 = b_pre.reshape(1, D_K), b_q.reshape(1, D_K), v_att.reshape(D_K, 1)
  nh = N // NSPLIT
  bh = B // NSPLIT
  bags = [
      _sc_gather(emb_table, ids3[k * bh:(k + 1) * bh].reshape(bh * W * NPB))
      for k in range(NSPLIT)
  ]
  ctx = None
  for k in range(NSPLIT):
    ctx = _tc_attention(bags[k], h3, W_pre, bp2, W_q, bq2, v2,
                        lens_node, sizes_node,
                        off_b=k * (bh // BPG), ctx_init=ctx)
  return (ctx.reshape(B, C, M, D_MODEL), hidden)


# speedup vs baseline: 1.1295x; 1.0112x over previous
import functools

import jax
import jax.numpy as jnp
from jax import lax
from jax.experimental import pallas as pl
from jax.experimental.pallas import tpu as pltpu
from jax.experimental.pallas import tpu_sc as plsc

B, C, M, W = 16, 4, 32, 16
D_MODEL = 256
D_K = 64
N = B * C * M
TOK = N * W

NC, NS = 2, 16
NW = NC * NS
CHUNK = 64
NBUF = 5


def _sc_gather(table, ids, num_cores=NC):
  ntok = ids.shape[0]
  nw = num_cores * NS
  tpw = ntok // nw
  cpw = tpw // CHUNK
  mesh = plsc.VectorSubcoreMesh(core_axis_name="c", subcore_axis_name="s",
                                num_cores=num_cores)

  @functools.partial(
      pl.kernel,
      mesh=mesh,
      out_type=jax.ShapeDtypeStruct((ntok, D_MODEL), jnp.float32),
      scratch_types=[
          pltpu.VMEM((tpw,), jnp.int32),
          pltpu.VMEM((NBUF, CHUNK, D_MODEL), jnp.float32),
      ] + [pltpu.SemaphoreType.DMA] * (2 * NBUF),
  )
  def gather_kernel(table_hbm, ids_hbm, out_hbm, idx_v, rows_v, *sems):
    CPW = cpw
    wid = lax.axis_index("s") * num_cores + lax.axis_index("c")
    base = wid * tpw
    pltpu.sync_copy(ids_hbm.at[pl.ds(base, tpw)], idx_v)
    gsems = list(sems[:NBUF])
    wsems = list(sems[NBUF:])

    def gstart(g):
      b = g % NBUF
      return pltpu.async_copy(
          table_hbm.at[idx_v.at[pl.ds(g * CHUNK, CHUNK)]],
          rows_v.at[b], gsems[b])

    gh = [None] * CPW
    wh = [None] * CPW
    for g in range(min(NBUF, CPW)):
      gh[g] = gstart(g)
    for g in range(CPW):
      b = g % NBUF
      gh[g].wait()
      wh[g] = pltpu.async_copy(
          rows_v.at[b], out_hbm.at[pl.ds(base + g * CHUNK, CHUNK)], wsems[b])
      if g + NBUF < CPW:
        wh[g].wait()
        gh[g + NBUF] = gstart(g + NBUF)
    for g in range(max(0, CPW - NBUF), CPW):
      wh[g].wait()

  return gather_kernel(table, ids)


NPB = N // B


BPG = 4


def _tc_body(bags_ref, h_ref, wp_ref, bp_ref, wq_ref, bq_ref, v_ref,
             lens_ref, szs_ref, out_ref):
  flat = bags_ref[...]
  pre = jnp.dot(flat.astype(jnp.bfloat16), wp_ref[...].astype(jnp.bfloat16),
                preferred_element_type=jnp.float32) + bp_ref[...]
  tpb = W * NPB
  for bb in range(BPG):
    q = jnp.dot(h_ref[bb], wq_ref[...],
                preferred_element_type=jnp.float32) + bq_ref[...]
    e = jnp.dot(jnp.tanh(pre[bb * tpb:(bb + 1) * tpb] + q), v_ref[...],
                preferred_element_type=jnp.float32)
    p = jnp.exp(e)
    lens = lens_ref[bb * NPB:(bb + 1) * NPB]
    ps = []
    denom = None
    for w in range(W):
      pw = p[w * NPB:(w + 1) * NPB] * (lens > float(w)).astype(jnp.float32)
      ps.append(pw)
      denom = pw if w == 0 else denom + pw
    mpos = lax.broadcasted_iota(jnp.int32, (NPB, 1), 0) % M
    nmask = (mpos.astype(jnp.float32) < szs_ref[bb * NPB:(bb + 1) * NPB]
             ).astype(jnp.float32)
    scale = nmask / denom
    base = bb * tpb
    ctx = ps[0] * scale * flat[base:base + NPB]
    for w in range(1, W):
      ctx = ctx + (ps[w] * scale) * flat[base + w * NPB:base + (w + 1) * NPB]
    out_ref[bb * NPB:(bb + 1) * NPB, :] = ctx


def _tc_attention(bags_t, hidden, W_pre, b_pre, W_q, b_q, v_att,
                  lens_node, sizes_node, off_b, ctx_init=None):
  nb = bags_t.shape[0] // (W * NPB)
  in_specs = [
      pl.BlockSpec((BPG * W * NPB, D_MODEL), lambda i: (i, 0)),
      pl.BlockSpec((BPG, 1, D_MODEL), lambda i: (i + off_b, 0, 0)),
      pl.BlockSpec((D_MODEL, D_K), lambda i: (0, 0)),
      pl.BlockSpec((1, D_K), lambda i: (0, 0)),
      pl.BlockSpec((D_MODEL, D_K), lambda i: (0, 0)),
      pl.BlockSpec((1, D_K), lambda i: (0, 0)),
      pl.BlockSpec((D_K, 1), lambda i: (0, 0)),
      pl.BlockSpec((BPG * NPB, 1), lambda i: (i + off_b, 0)),
      pl.BlockSpec((BPG * NPB, 1), lambda i: (i + off_b, 0)),
  ]
  args = [bags_t, hidden, W_pre, b_pre, W_q, b_q, v_att,
          lens_node, sizes_node]
  body = _tc_body
  aliases = {}
  if ctx_init is not None:
    in_specs.append(pl.BlockSpec(memory_space=pltpu.MemorySpace.HBM))
    args.append(ctx_init)
    aliases = {9: 0}
    body = lambda *refs: _tc_body(*refs[:9], refs[10])
  return pl.pallas_call(
      body,
      grid=(nb // BPG,),
      in_specs=in_specs,
      out_specs=pl.BlockSpec((BPG * NPB, D_MODEL), lambda i: (i + off_b, 0)),
      out_shape=jax.ShapeDtypeStruct((N, D_MODEL), jnp.float32),
      input_output_aliases=aliases,
  )(*args)


NSPLIT = 2


def kernel(con_hidden, emb_table, W_pre, b_pre, W_q, b_q, v_att,
           token_ids, node_lengths, node_sizes):
  hidden = jnp.concatenate([con_hidden[0], con_hidden[1]], axis=1)
  ids3 = token_ids.reshape(B, NPB, W).swapaxes(1, 2)
  lens_node = node_lengths.astype(jnp.float32).reshape(N, 1)
  sizes_node = jnp.repeat(node_sizes, M).astype(jnp.float32).reshape(N, 1)
  h3 = hidden.reshape(B, 1, D_MODEL)
  bp2, bq2, v2 = b_pre.reshape(1, D_K), b_q.reshape(1, D_K), v_att.reshape(D_K, 1)
  nh = N // NSPLIT
  bh = B // NSPLIT
  bags = [
      _sc_gather(emb_table, ids3[k * bh:(k + 1) * bh].reshape(bh * W * NPB))
      for k in range(NSPLIT)
  ]
  ctx = None
  for k in range(NSPLIT):
    ctx = _tc_attention(bags[k], h3, W_pre, bp2, W_q, bq2, v2,
                        lens_node, sizes_node,
                        off_b=k * (bh // BPG), ctx_init=ctx)
  return (ctx.reshape(B, C, M, D_MODEL), hidden)
